# baseline (speedup 1.0000x reference)
import functools

import jax
import jax.numpy as jnp
from jax import lax
from jax.experimental import pallas as pl
from jax.experimental.pallas import tpu as pltpu
from jax.experimental.pallas import tpu_sc as plsc

_NUM_SC = 2
_NUM_SUBCORES = 16
_NUM_WORKERS = _NUM_SC * _NUM_SUBCORES


def _make_sc_gather(n_tokens: int, embed: int, chunk: int):
    per_worker = n_tokens // _NUM_WORKERS
    assert per_worker % chunk == 0
    n_chunks = per_worker // chunk
    mesh = plsc.VectorSubcoreMesh(core_axis_name="c", subcore_axis_name="s")

    nbuf = 3
    assert n_chunks >= nbuf

    @functools.partial(
        pl.kernel,
        mesh=mesh,
        out_type=jax.ShapeDtypeStruct((n_tokens, embed), jnp.float32),
        scratch_types=[
            pltpu.VMEM((per_worker,), jnp.int32),
            [pltpu.VMEM((chunk, embed), jnp.float32)] * nbuf,
            [pltpu.SemaphoreType.DMA] * nbuf,
            [pltpu.SemaphoreType.DMA] * nbuf,
        ],
    )
    def gather(table_hbm, idx_hbm, out_hbm, idx_v, bufs, gsems, osems):
        wid = lax.axis_index("s") * _NUM_SC + lax.axis_index("c")
        base = wid * per_worker
        pltpu.sync_copy(idx_hbm.at[pl.ds(base, per_worker)], idx_v)

        def start_gather(c):
            return pltpu.async_copy(
                table_hbm.at[idx_v.at[pl.ds(c * chunk, chunk)]],
                bufs[c % nbuf],
                gsems[c % nbuf],
            )

        def start_out(c):
            return pltpu.async_copy(
                bufs[c % nbuf],
                out_hbm.at[pl.ds(base + c * chunk, chunk)],
                osems[c % nbuf],
            )

        gathers = [None] * n_chunks
        outs = [None] * n_chunks
        for c in range(nbuf - 1):
            gathers[c] = start_gather(c)
        for c in range(n_chunks):
            gathers[c].wait()
            outs[c] = start_out(c)
            if c + nbuf - 1 < n_chunks:
                if c >= 1:
                    outs[c - 1].wait()
                gathers[c + nbuf - 1] = start_gather(c + nbuf - 1)
        for c in range(n_chunks - nbuf, n_chunks):
            if outs[c] is not None and c >= 0:
                outs[c].wait()

    return gather


def _ln_matmul_body(x_ref, pos_ref, g_ref, bt_ref, w_ref, bias_ref, o_ref):
    x = x_ref[...] + pos_ref[...]
    mu = jnp.mean(x, axis=-1, keepdims=True)
    xc = x - mu
    var = jnp.mean(xc * xc, axis=-1, keepdims=True)
    xn = xc * lax.rsqrt(var + 1e-12)
    xn = xn * g_ref[...] + bt_ref[...]
    o_ref[...] = (
        jnp.dot(
            xn,
            w_ref[...],
            preferred_element_type=jnp.float32,
        )
        + bias_ref[...]
    )


def _make_tc_fused(n_tokens: int, seq: int, embed: int, hidden: int, tm: int):
    n_batch = n_tokens // seq
    pos_blocks = seq // tm
    grid = (pos_blocks, n_batch)

    return pl.pallas_call(
        _ln_matmul_body,
        grid=grid,
        in_specs=[
            pl.BlockSpec((tm, embed), lambda p, j: (j * pos_blocks + p, 0)),
            pl.BlockSpec((tm, embed), lambda p, j: (p, 0)),
            pl.BlockSpec((1, embed), lambda p, j: (0, 0)),
            pl.BlockSpec((1, embed), lambda p, j: (0, 0)),
            pl.BlockSpec((embed, hidden), lambda p, j: (0, 0)),
            pl.BlockSpec((1, hidden), lambda p, j: (0, 0)),
        ],
        out_specs=pl.BlockSpec((tm, hidden), lambda p, j: (j * pos_blocks + p, 0)),
        out_shape=jax.ShapeDtypeStruct((n_tokens, hidden), jnp.float32),
    )


def kernel(input_ids, word_table, pos_table, ln_gamma, ln_beta, W, b):
    bsz, seq = input_ids.shape
    vocab, embed = word_table.shape
    hidden = W.shape[1]
    n_tokens = bsz * seq

    ids_flat = input_ids.reshape(n_tokens).astype(jnp.int32)

    gathered = _make_sc_gather(n_tokens, embed, chunk=32)(word_table, ids_flat)
    fused = _make_tc_fused(n_tokens, seq, embed, hidden, tm=1024)
    out = fused(
        gathered,
        pos_table[:seq],
        ln_gamma.reshape(1, embed),
        ln_beta.reshape(1, embed),
        W,
        b.reshape(1, hidden),
    )
    return out.reshape(bsz, seq, hidden)


# speedup vs baseline: 1.0221x; 1.0221x over previous
import functools

import jax
import jax.numpy as jnp
from jax import lax
from jax.experimental import pallas as pl
from jax.experimental.pallas import tpu as pltpu
from jax.experimental.pallas import tpu_sc as plsc

_NUM_SC = 2
_NUM_SUBCORES = 16
_NUM_WORKERS = _NUM_SC * _NUM_SUBCORES


def _make_sc_gather(n_tokens: int, embed: int, chunk: int):
    per_worker = n_tokens // _NUM_WORKERS
    assert per_worker % chunk == 0
    n_chunks = per_worker // chunk
    mesh = plsc.VectorSubcoreMesh(core_axis_name="c", subcore_axis_name="s")

    nbuf = 6
    assert n_chunks >= nbuf

    @functools.partial(
        pl.kernel,
        mesh=mesh,
        out_type=jax.ShapeDtypeStruct((n_tokens, embed), jnp.float32),
        scratch_types=[
            pltpu.VMEM((per_worker,), jnp.int32),
            [pltpu.VMEM((chunk, embed), jnp.float32)] * nbuf,
            [pltpu.SemaphoreType.DMA] * nbuf,
            [pltpu.SemaphoreType.DMA] * nbuf,
        ],
    )
    def gather(table_hbm, idx_hbm, out_hbm, idx_v, bufs, gsems, osems):
        wid = lax.axis_index("s") * _NUM_SC + lax.axis_index("c")
        base = wid * per_worker
        pltpu.sync_copy(idx_hbm.at[pl.ds(base, per_worker)], idx_v)

        def start_gather(c):
            return pltpu.async_copy(
                table_hbm.at[idx_v.at[pl.ds(c * chunk, chunk)]],
                bufs[c % nbuf],
                gsems[c % nbuf],
            )

        def start_out(c):
            return pltpu.async_copy(
                bufs[c % nbuf],
                out_hbm.at[pl.ds(base + c * chunk, chunk)],
                osems[c % nbuf],
            )

        gathers = [None] * n_chunks
        outs = [None] * n_chunks
        for c in range(nbuf - 1):
            gathers[c] = start_gather(c)
        for c in range(n_chunks):
            gathers[c].wait()
            outs[c] = start_out(c)
            if c + nbuf - 1 < n_chunks:
                if c >= 1:
                    outs[c - 1].wait()
                gathers[c + nbuf - 1] = start_gather(c + nbuf - 1)
        for c in range(n_chunks - nbuf, n_chunks):
            if outs[c] is not None and c >= 0:
                outs[c].wait()

    return gather


def _ln_matmul_body(x_ref, pos_ref, g_ref, bt_ref, w_ref, bias_ref, o_ref):
    x = x_ref[...] + pos_ref[...]
    mu = jnp.mean(x, axis=-1, keepdims=True)
    xc = x - mu
    var = jnp.mean(xc * xc, axis=-1, keepdims=True)
    xn = xc * lax.rsqrt(var + 1e-12)
    xn = xn * g_ref[...] + bt_ref[...]
    o_ref[...] = (
        jnp.dot(
            xn,
            w_ref[...],
            preferred_element_type=jnp.float32,
        )
        + bias_ref[...]
    )


def _make_tc_fused(n_tokens: int, seq: int, embed: int, hidden: int, tm: int):
    n_batch = n_tokens // seq
    pos_blocks = seq // tm
    grid = (pos_blocks, n_batch)

    return pl.pallas_call(
        _ln_matmul_body,
        grid=grid,
        in_specs=[
            pl.BlockSpec((tm, embed), lambda p, j: (j * pos_blocks + p, 0)),
            pl.BlockSpec((tm, embed), lambda p, j: (p, 0)),
            pl.BlockSpec((1, embed), lambda p, j: (0, 0)),
            pl.BlockSpec((1, embed), lambda p, j: (0, 0)),
            pl.BlockSpec((embed, hidden), lambda p, j: (0, 0)),
            pl.BlockSpec((1, hidden), lambda p, j: (0, 0)),
        ],
        out_specs=pl.BlockSpec((tm, hidden), lambda p, j: (j * pos_blocks + p, 0)),
        out_shape=jax.ShapeDtypeStruct((n_tokens, hidden), jnp.float32),
    )


def kernel(input_ids, word_table, pos_table, ln_gamma, ln_beta, W, b):
    bsz, seq = input_ids.shape
    vocab, embed = word_table.shape
    hidden = W.shape[1]
    n_tokens = bsz * seq

    ids_flat = input_ids.reshape(n_tokens).astype(jnp.int32)

    gathered = _make_sc_gather(n_tokens, embed, chunk=16)(word_table, ids_flat)
    fused = _make_tc_fused(n_tokens, seq, embed, hidden, tm=1024)
    out = fused(
        gathered,
        pos_table[:seq],
        ln_gamma.reshape(1, embed),
        ln_beta.reshape(1, embed),
        W,
        b.reshape(1, hidden),
    )
    return out.reshape(bsz, seq, hidden)
